# baseline (speedup 1.0000x reference)
import functools

import jax
import jax.numpy as jnp
from jax import lax
from jax.experimental import pallas as pl
from jax.experimental.pallas import tpu as pltpu
from jax.experimental.pallas import tpu_sc as plsc

_B, _T, _D = 16, 2048, 1024
_L16 = 16
_CH = 8
_NC, _NS = 2, 16
_NW = _NC * _NS
_NSLICE = _D // _L16

_mesh = plsc.VectorSubcoreMesh(core_axis_name="c", subcore_axis_name="s")


@functools.partial(
    pl.kernel,
    out_type=jax.ShapeDtypeStruct((_NC, _NS, _D), jnp.float32),
    mesh=_mesh,
    scratch_types=[
        pltpu.VMEM((_CH, _D), jnp.float32),
        pltpu.VMEM((_B * _D,), jnp.float32),
        pltpu.VMEM((_NS, _D), jnp.float32),
        pltpu.VMEM((_D,), jnp.float32),
        pltpu.VMEM((2 * _L16,), jnp.int32),
        pltpu.VMEM_SHARED((_B, _NS, _D), jnp.float32),
    ],
)
def _sc_rowsum(payload, seq_lens, out, buf, part_v, gbuf, out_v, lens_v,
               shared):
    cid = lax.axis_index("c")
    sid = lax.axis_index("s")
    gwid = cid * _NS + sid

    pltpu.sync_copy(seq_lens, lens_v.at[pl.ds(0, _B)])

    def seq_body(i, _):
        ibase = i * _D

        def zbody(jj, _):
            part_v[pl.ds(ibase + jj * _L16, _L16)] = jnp.zeros(
                (_L16,), jnp.float32)
            return 0
        lax.fori_loop(0, _NSLICE, zbody, 0)

        L = lens_v[pl.ds(i, _L16)][0]
        nblk = (L + (_CH - 1)) // _CH
        cnt = (nblk - gwid + (_NW - 1)) // _NW

        def chunk_body(c, _):
            start = (gwid + c * _NW) * _CH
            pltpu.sync_copy(payload.at[i, pl.ds(start, _CH), :], buf)
            nv = jnp.minimum(L - start, _CH)

            def row_body(r, _):
                for jj in range(_NSLICE):
                    sl = pl.ds(ibase + jj * _L16, _L16)
                    part_v[sl] = part_v[sl] + buf[r, pl.ds(jj * _L16, _L16)]
                return 0
            lax.fori_loop(0, nv, row_body, 0)
            return 0
        lax.fori_loop(0, cnt, chunk_body, 0)

        pltpu.sync_copy(part_v.at[pl.ds(ibase, _D)], shared.at[i, sid])
        return 0
    lax.fori_loop(0, _B, seq_body, 0)

    plsc.subcore_barrier()

    pltpu.sync_copy(shared.at[sid], gbuf)

    def rbody(jj, _):
        sl = pl.ds(jj * _L16, _L16)
        s = jnp.zeros((_L16,), jnp.float32)
        for w in range(_NS):
            s = s + gbuf[w, sl]
        out_v[sl] = s
        return 0
    lax.fori_loop(0, _NSLICE, rbody, 0)

    pltpu.sync_copy(out_v, out.at[cid, sid])


def _tc_epilogue(part_ref, w_ref, b_ref, lens_ref, out_ref):
    wcol = jnp.sum(w_ref[...], axis=0, keepdims=True)
    rs = part_ref[0] + part_ref[1]
    s = jnp.sum(rs * wcol, axis=1)
    lens_f = lens_ref[...].reshape(_B).astype(jnp.float32)
    bmean = jnp.sum(b_ref[...]) * (1.0 / _D)
    means = s / (lens_f * float(_D)) + bmean
    out_ref[...] = means.reshape(1, _B)


def kernel(payload, seq_lens, W, b):
    partials = _sc_rowsum(payload, seq_lens)
    means2d = pl.pallas_call(
        _tc_epilogue,
        out_shape=jax.ShapeDtypeStruct((1, _B), jnp.float32),
    )(partials, W, b.reshape(1, _D), seq_lens.reshape(1, _B))
    return means2d.reshape(_B)


# speedup vs baseline: 1.1452x; 1.1452x over previous
import functools

import jax
import jax.numpy as jnp
from jax import lax
from jax.experimental import pallas as pl
from jax.experimental.pallas import tpu as pltpu
from jax.experimental.pallas import tpu_sc as plsc

_B, _T, _D = 16, 2048, 1024
_L16 = 16
_CH = 8
_NC, _NS = 2, 16
_NW = _NC * _NS
_NSLICE = _D // _L16
_NBUF = 3

_mesh = plsc.VectorSubcoreMesh(core_axis_name="c", subcore_axis_name="s")


@functools.partial(
    pl.kernel,
    out_type=jax.ShapeDtypeStruct((_NC, _NS, _D), jnp.float32),
    mesh=_mesh,
    scratch_types=[
        pltpu.VMEM((_NBUF, _CH, _D), jnp.float32),
        pltpu.VMEM((_B * _D,), jnp.float32),
        pltpu.VMEM((_NS, _D), jnp.float32),
        pltpu.VMEM((_D,), jnp.float32),
        pltpu.VMEM((2 * _L16,), jnp.int32),
        pltpu.VMEM_SHARED((_B, _NS, _D), jnp.float32),
        pltpu.SemaphoreType.DMA,
        pltpu.SemaphoreType.DMA,
        pltpu.SemaphoreType.DMA,
    ],
)
def _sc_rowsum(payload, seq_lens, out, buf, part_v, gbuf, out_v, lens_v,
               shared, sem0, sem1, sem2):
    sems = (sem0, sem1, sem2)
    cid = lax.axis_index("c")
    sid = lax.axis_index("s")
    gwid = cid * _NS + sid

    pltpu.sync_copy(seq_lens, lens_v.at[pl.ds(0, _B)])

    def seq_body(i, _):
        ibase = i * _D

        def zbody(jj, _):
            part_v[pl.ds(ibase + jj * _L16, _L16)] = jnp.zeros(
                (_L16,), jnp.float32)
            return 0
        lax.fori_loop(0, _NSLICE, zbody, 0)

        L = lens_v[pl.ds(i, _L16)][0]
        nblk = (L + (_CH - 1)) // _CH
        lo = (gwid * nblk) // _NW
        cnt = ((gwid + 1) * nblk) // _NW - lo

        def issue(c, b):
            start = (lo + c) * _CH
            pltpu.async_copy(payload.at[i, pl.ds(start, _CH), :],
                             buf.at[b], sems[b])

        for b in range(_NBUF):
            @pl.when(b < cnt)
            def _():
                issue(jnp.int32(b), b)

        def ring_body(g, _):
            for b in range(_NBUF):
                c = g * _NBUF + b

                @pl.when(c < cnt)
                def _():
                    pltpu.make_async_copy(payload.at[0, pl.ds(0, _CH), :],
                                          buf.at[b], sems[b]).wait()
                    start = (lo + c) * _CH
                    nv = L - start
                    nvv = jnp.broadcast_to(nv, (_L16,))
                    fvs = [jnp.where(jnp.full((_L16,), r, jnp.int32) < nvv,
                                     1.0, 0.0).astype(jnp.float32)
                           for r in range(_CH)]
                    for jj in range(_NSLICE):
                        sl = pl.ds(ibase + jj * _L16, _L16)
                        acc = part_v[sl]
                        for r in range(_CH):
                            acc = acc + buf[b, r, pl.ds(jj * _L16,
                                                        _L16)] * fvs[r]
                        part_v[sl] = acc

                    @pl.when(c + _NBUF < cnt)
                    def _():
                        issue(c + _NBUF, b)
            return 0
        lax.fori_loop(0, (cnt + (_NBUF - 1)) // _NBUF, ring_body, 0)

        pltpu.sync_copy(part_v.at[pl.ds(ibase, _D)], shared.at[i, sid])
        return 0
    lax.fori_loop(0, _B, seq_body, 0)

    plsc.subcore_barrier()

    pltpu.sync_copy(shared.at[sid], gbuf)

    def rbody(jj, _):
        sl = pl.ds(jj * _L16, _L16)
        s = jnp.zeros((_L16,), jnp.float32)
        for w in range(_NS):
            s = s + gbuf[w, sl]
        out_v[sl] = s
        return 0
    lax.fori_loop(0, _NSLICE, rbody, 0)

    pltpu.sync_copy(out_v, out.at[cid, sid])


def _tc_epilogue(part_ref, w_ref, b_ref, lens_ref, out_ref):
    wcol = jnp.sum(w_ref[...], axis=0, keepdims=True)
    rs = part_ref[0] + part_ref[1]
    s = jnp.sum(rs * wcol, axis=1)
    lens_f = lens_ref[...].reshape(_B).astype(jnp.float32)
    bmean = jnp.sum(b_ref[...]) * (1.0 / _D)
    means = s / (lens_f * float(_D)) + bmean
    out_ref[...] = means.reshape(1, _B)


def kernel(payload, seq_lens, W, b):
    partials = _sc_rowsum(payload, seq_lens)
    means2d = pl.pallas_call(
        _tc_epilogue,
        out_shape=jax.ShapeDtypeStruct((1, _B), jnp.float32),
    )(partials, W, b.reshape(1, _D), seq_lens.reshape(1, _B))
    return means2d.reshape(_B)
